# baseline (speedup 1.0000x reference)
import functools

import jax
import jax.numpy as jnp
from jax import lax
from jax.experimental import pallas as pl
from jax.experimental.pallas import tpu as pltpu
from jax.experimental.pallas import tpu_sc as plsc

N = 10000
E = 320000
D = 128
L = 4
Q = 3
K = 16
G = 64
DOUT = 10

RB = 2000
GRID = N // RB

NC, NS = 2, 16
NW = NC * NS
ECH = 128
EPT = 10240
EPAD = EPT * NW
NCH = EPT // ECH
NCH0 = 80
NCH1 = 0
NPH = 40
PR = 2
DGRP = 8
NPAD = 10240
ROWS_T = NPAD // NS
DEGC = 128


def _sc_mesh():
    return plsc.VectorSubcoreMesh(core_axis_name="c", subcore_axis_name="s")


def _deg_partials(dst_pad, ones_hbm, zeros16_hbm):
    @functools.partial(
        pl.kernel,
        out_type=jax.ShapeDtypeStruct((NC, NPAD, DEGC), jnp.float32),
        mesh=_sc_mesh(),
        scratch_types=[
            pltpu.VMEM((NCH, ECH), jnp.int32),
            pltpu.VMEM((ECH, DEGC), jnp.float32),
            pltpu.VMEM_SHARED((NPAD, DEGC), jnp.float32),
            pltpu.SemaphoreType.DMA,
        ],
    )
    def k(dst_hbm, ones_ref, zeros_ref, out_hbm, didx2, ones_v, acc, sem):
        c = lax.axis_index("c")
        s = lax.axis_index("s")
        wid = s * NC + c
        pltpu.sync_copy(ones_ref, ones_v)
        pltpu.sync_copy(dst_hbm.at[pl.ds(wid * NCH, NCH)], didx2)
        pltpu.sync_copy(zeros_ref, acc.at[pl.ds(s * ROWS_T, ROWS_T)])
        plsc.subcore_barrier()

        def group(jo, carry):
            for b in range(DGRP):
                j = jo * DGRP + b
                pltpu.async_copy(ones_v, acc.at[didx2.at[j]], sem, add=True)
            for b in range(DGRP):
                pltpu.make_async_copy(ones_ref, ones_v, sem).wait()
            return carry

        lax.fori_loop(0, NCH // DGRP, group, 0)
        plsc.subcore_barrier()
        pltpu.sync_copy(acc.at[pl.ds(s * ROWS_T, ROWS_T)],
                        out_hbm.at[c].at[pl.ds(s * ROWS_T, ROWS_T)])

    return k(dst_pad, ones_hbm, zeros16_hbm)


def _agg_partials(ht, src_pad, dst_pad, zeros_hbm):
    @functools.partial(
        pl.kernel,
        out_type=jax.ShapeDtypeStruct((NPAD, D), jnp.float32),
        mesh=_sc_mesh(),
        scratch_types=[
            pltpu.VMEM((NPH, ECH), jnp.int32),
            pltpu.VMEM((NPH, ECH), jnp.int32),
            pltpu.VMEM((PR, ECH, D), jnp.float32),
            pltpu.VMEM_SHARED((NPAD, D), jnp.float32),
            pltpu.SemaphoreType.DMA((PR,)),
        ],
    )
    def k(ht_hbm, src_hbm, dst_hbm, zeros_ref, out_hbm,
          sidx2, didx2, rows, acc, gsem):
        c = lax.axis_index("c")
        s = lax.axis_index("s")

        @pl.when(c == 0)
        def _():
            pltpu.sync_copy(zeros_ref, acc.at[pl.ds(s * ROWS_T, ROWS_T)])

        plsc.subcore_barrier()

        nphases = jnp.where(c == 0, NCH0 // NPH, 0)
        cbase = s * NCH0

        def gwait(b):
            pltpu.make_async_copy(ht_hbm.at[pl.ds(0, ECH)], rows.at[b],
                                  gsem.at[b]).wait()

        def phase(ph, carry):
            base = cbase + ph * NPH
            pltpu.sync_copy(src_hbm.at[pl.ds(base, NPH)], sidx2)
            pltpu.sync_copy(dst_hbm.at[pl.ds(base, NPH)], didx2)
            for b in range(PR):
                pltpu.async_copy(ht_hbm.at[sidx2.at[b]], rows.at[b],
                                 gsem.at[b])

            def inner(jo, carry2):
                for b in range(PR):
                    j = jo * PR + b
                    gwait(b)
                    pltpu.sync_copy(rows.at[b], acc.at[didx2.at[j]], add=True)
                    jn = jnp.minimum(j + PR, NPH - 1)
                    pltpu.async_copy(ht_hbm.at[sidx2.at[jn]], rows.at[b],
                                     gsem.at[b])
                return carry2

            lax.fori_loop(0, NPH // PR, inner, 0)
            for b in range(PR):
                gwait(b)
            return carry

        lax.fori_loop(0, nphases, phase, 0)
        plsc.subcore_barrier()

        @pl.when(c == 0)
        def _():
            pltpu.sync_copy(acc.at[pl.ds(s * ROWS_T, ROWS_T)],
                            out_hbm.at[pl.ds(s * ROWS_T, ROWS_T)])

    return k(ht, src_pad, dst_pad, zeros_hbm)


def _enc_mm_body(x_ref, we_ref, be_ref, h_ref):
    h_ref[...] = jnp.dot(x_ref[...], we_ref[...],
                         preferred_element_type=jnp.float32) + be_ref[...]


def _enc_mm(x, W_enc, b_enc2):
    return pl.pallas_call(
        _enc_mm_body,
        grid=(GRID,),
        in_specs=[
            pl.BlockSpec((RB, D), lambda i: (i, 0)),
            pl.BlockSpec((D, D), lambda i: (0, 0)),
            pl.BlockSpec((1, D), lambda i: (0, 0)),
        ],
        out_specs=pl.BlockSpec((RB, D), lambda i: (i, 0)),
        out_shape=jax.ShapeDtypeStruct((N, D), jnp.float32),
    )(x, W_enc, b_enc2)


def _enc_scale_body(h_ref, dga_ref, dgb_ref, ht_ref, dv_ref):
    deg = dga_ref[:, 0:1] + dgb_ref[:, 0:1]
    dinv = 1.0 / jnp.sqrt(jnp.maximum(deg, 1.0))
    dinvb = jnp.broadcast_to(dinv, (RB, D))
    dv_ref[...] = dinvb
    ht_ref[...] = h_ref[...] * dinvb


def _enc_scale(h, degA, degB):
    return pl.pallas_call(
        _enc_scale_body,
        grid=(GRID,),
        in_specs=[
            pl.BlockSpec((RB, D), lambda i: (i, 0)),
            pl.BlockSpec((RB, DEGC), lambda i: (i, 0)),
            pl.BlockSpec((RB, DEGC), lambda i: (i, 0)),
        ],
        out_specs=[
            pl.BlockSpec((RB, D), lambda i: (i, 0)),
            pl.BlockSpec((RB, D), lambda i: (i, 0)),
        ],
        out_shape=[
            jax.ShapeDtypeStruct((N, D), jnp.float32),
            jax.ShapeDtypeStruct((N, D), jnp.float32),
        ],
    )(h, degA, degB)


def _conv_body(s0_ref, s1_ref, h_ref, dv_ref, w_ref, b_ref, hn_ref, htn_ref):
    dinvb = dv_ref[...]
    agg = (s0_ref[...] + s1_ref[...]) * dinvb
    z = jnp.dot(agg, w_ref[...], preferred_element_type=jnp.float32) + b_ref[...]
    h = jnp.maximum(z, 0.0) + h_ref[...]
    hn_ref[...] = h
    htn_ref[...] = h * dinvb


def _conv(S0, S1, h, dinvb, W_l, b_l2):
    return pl.pallas_call(
        _conv_body,
        grid=(GRID,),
        in_specs=[
            pl.BlockSpec((RB, D), lambda i: (i, 0)),
            pl.BlockSpec((RB, D), lambda i: (i, 0)),
            pl.BlockSpec((RB, D), lambda i: (i, 0)),
            pl.BlockSpec((RB, D), lambda i: (i, 0)),
            pl.BlockSpec((D, D), lambda i: (0, 0)),
            pl.BlockSpec((1, D), lambda i: (0, 0)),
        ],
        out_specs=[
            pl.BlockSpec((RB, D), lambda i: (i, 0)),
            pl.BlockSpec((RB, D), lambda i: (i, 0)),
        ],
        out_shape=[
            jax.ShapeDtypeStruct((N, D), jnp.float32),
            jax.ShapeDtypeStruct((N, D), jnp.float32),
        ],
    )(S0, S1, h, dinvb, W_l, b_l2)


def _vq_body(h_ref, cb_ref, ids_ref, com_ref):
    i = pl.program_id(0)

    @pl.when(i == 0)
    def _():
        com_ref[...] = jnp.zeros((1, 1), jnp.float32)

    resid = h_ref[...]
    kio = lax.broadcasted_iota(jnp.int32, (RB, K), 1)
    csum = jnp.float32(0.0)
    idxs = []
    for q in range(Q):
        cb = cb_ref[q]
        nrm = jnp.sqrt(jnp.sum(cb * cb, axis=1, keepdims=True)) + 1e-8
        cbn = cb / nrm
        sim = lax.dot_general(resid, cbn, (((1,), (1,)), ((), ())),
                              preferred_element_type=jnp.float32)
        mx = jnp.max(sim, axis=1, keepdims=True)
        idx = jnp.min(jnp.where(sim >= mx, kio, K), axis=1, keepdims=True)
        onehot = (kio == idx).astype(jnp.float32)
        quant = jnp.dot(onehot, cbn, preferred_element_type=jnp.float32)
        diff = quant - resid
        csum = csum + jnp.sum(diff * diff)
        resid = resid - quant
        idxs.append(idx.astype(jnp.float32))
    ids_ref[...] = jnp.concatenate(idxs, axis=1)
    com_ref[...] = com_ref[...] + csum * (0.25 / (N * D))


def _vq(h, cb_l):
    return pl.pallas_call(
        _vq_body,
        grid=(GRID,),
        in_specs=[
            pl.BlockSpec((RB, D), lambda i: (i, 0)),
            pl.BlockSpec((Q, K, D), lambda i: (0, 0, 0)),
        ],
        out_specs=[
            pl.BlockSpec((RB, Q), lambda i: (i, 0)),
            pl.BlockSpec((1, 1), lambda i: (0, 0)),
        ],
        out_shape=[
            jax.ShapeDtypeStruct((N, Q), jnp.float32),
            jax.ShapeDtypeStruct((1, 1), jnp.float32),
        ],
    )(h, cb_l)


def _pool_body(h_ref, ids_ref, bi_ref, wh_ref, bh_ref,
               out_ref, gid_ref, pacc, gacc, cacc):
    i = pl.program_id(0)

    @pl.when(i == 0)
    def _():
        pacc[...] = jnp.zeros_like(pacc)
        gacc[...] = jnp.zeros_like(gacc)
        cacc[...] = jnp.zeros_like(cacc)

    gio = lax.broadcasted_iota(jnp.int32, (RB, G), 1)
    oh = (bi_ref[...] == gio).astype(jnp.float32)
    pacc[...] += lax.dot_general(oh, h_ref[...], (((0,), (0,)), ((), ())),
                                 preferred_element_type=jnp.float32)
    gacc[...] += lax.dot_general(oh, ids_ref[...], (((0,), (0,)), ((), ())),
                                 preferred_element_type=jnp.float32)
    ones = jnp.ones((RB, 1), jnp.float32)
    cacc[...] += lax.dot_general(oh, ones, (((0,), (0,)), ((), ())),
                                 preferred_element_type=jnp.float32)

    @pl.when(i == pl.num_programs(0) - 1)
    def _():
        pooled = pacc[...] / jnp.maximum(cacc[...], 1.0)
        out_ref[...] = jnp.dot(pooled, wh_ref[...],
                               preferred_element_type=jnp.float32) + bh_ref[...]
        gid_ref[...] = gacc[...]


def _pool(h, ids_cat, bi2, W_head, b_head2):
    return pl.pallas_call(
        _pool_body,
        grid=(GRID,),
        in_specs=[
            pl.BlockSpec((RB, D), lambda i: (i, 0)),
            pl.BlockSpec((RB, L * Q), lambda i: (i, 0)),
            pl.BlockSpec((RB, 1), lambda i: (i, 0)),
            pl.BlockSpec((D, DOUT), lambda i: (0, 0)),
            pl.BlockSpec((1, DOUT), lambda i: (0, 0)),
        ],
        out_specs=[
            pl.BlockSpec((G, DOUT), lambda i: (0, 0)),
            pl.BlockSpec((G, L * Q), lambda i: (0, 0)),
        ],
        out_shape=[
            jax.ShapeDtypeStruct((G, DOUT), jnp.float32),
            jax.ShapeDtypeStruct((G, L * Q), jnp.float32),
        ],
        scratch_shapes=[
            pltpu.VMEM((G, D), jnp.float32),
            pltpu.VMEM((G, L * Q), jnp.float32),
            pltpu.VMEM((G, 1), jnp.float32),
        ],
    )(h, ids_cat, bi2, W_head, b_head2)


def kernel(x, edge_index, batch_idx, W_enc, b_enc, W_conv, b_conv,
           codebooks, W_head, b_head):
    src = edge_index[0]
    dst = edge_index[1]
    pad_e = EPAD - E
    src_pad = jnp.concatenate([src, jnp.zeros((pad_e,), jnp.int32)]).reshape(NW * NCH, ECH)
    dst_pad = jnp.concatenate([dst, jnp.full((pad_e,), N, jnp.int32)]).reshape(NW * NCH, ECH)
    ones_hbm = jnp.ones((ECH, DEGC), jnp.float32)
    zeros16_hbm = jnp.zeros((ROWS_T, DEGC), jnp.float32)
    zeros_hbm = jnp.zeros((ROWS_T, D), jnp.float32)

    degP = _deg_partials(dst_pad, ones_hbm, zeros16_hbm)
    h = _enc_mm(x, W_enc, b_enc[None, :])
    ht, dinvb = _enc_scale(h, degP[0, :N], degP[1, :N])

    half = NS * NCH0
    src_a, src_b = src_pad[:half], src_pad[half:]
    dst_a, dst_b = dst_pad[:half], dst_pad[half:]

    commits = []
    ids_list = []
    for l in range(L):
        SPa = _agg_partials(ht, src_a, dst_a, zeros_hbm)
        SPb = _agg_partials(ht, src_b, dst_b, zeros_hbm)
        h, ht = _conv(SPa, SPb, h, dinvb,
                      W_conv[l], b_conv[l][None, :])
        ids_l, com_l = _vq(h, codebooks[l])
        commits.append(com_l)
        ids_list.append(ids_l)

    ids_cat = jnp.concatenate(ids_list, axis=1)
    out, graph_id = _pool(h, ids_cat, batch_idx[:, None],
                          W_head, b_head[None, :])
    total_commit = (commits[0] + commits[1] + commits[2] + commits[3])[0, 0]
    return out, total_commit, graph_id


# speedup vs baseline: 1.3723x; 1.3723x over previous
import functools

import jax
import jax.numpy as jnp
from jax import lax
from jax.experimental import pallas as pl
from jax.experimental.pallas import tpu as pltpu
from jax.experimental.pallas import tpu_sc as plsc

N = 10000
E = 320000
D = 128
L = 4
Q = 3
K = 16
G = 64
DOUT = 10

RB = 2000
GRID = N // RB

NC, NS = 2, 16
NW = NC * NS
ECH = 128
EPT = 10240
EPAD = EPT * NW
NCH = EPT // ECH
NCH0 = 80
NCH1 = 0
NPH = 40
PR = 2
DGRP = 8
NPAD = 10240
ROWS_T = NPAD // NS
DEGC = 128


def _sc_mesh():
    return plsc.VectorSubcoreMesh(core_axis_name="c", subcore_axis_name="s")


def _deg_partials(dst_pad, ones_hbm, zeros16_hbm):
    @functools.partial(
        pl.kernel,
        out_type=jax.ShapeDtypeStruct((NC, NPAD, DEGC), jnp.float32),
        mesh=_sc_mesh(),
        scratch_types=[
            pltpu.VMEM((NCH, ECH), jnp.int32),
            pltpu.VMEM((ECH, DEGC), jnp.float32),
            pltpu.VMEM_SHARED((NPAD, DEGC), jnp.float32),
            pltpu.SemaphoreType.DMA,
        ],
    )
    def k(dst_hbm, ones_ref, zeros_ref, out_hbm, didx2, ones_v, acc, sem):
        c = lax.axis_index("c")
        s = lax.axis_index("s")
        wid = s * NC + c
        pltpu.sync_copy(ones_ref, ones_v)
        pltpu.sync_copy(dst_hbm.at[pl.ds(wid * NCH, NCH)], didx2)
        pltpu.sync_copy(zeros_ref, acc.at[pl.ds(s * ROWS_T, ROWS_T)])
        plsc.subcore_barrier()

        def group(jo, carry):
            for b in range(DGRP):
                j = jo * DGRP + b
                pltpu.async_copy(ones_v, acc.at[didx2.at[j]], sem, add=True)
            for b in range(DGRP):
                pltpu.make_async_copy(ones_ref, ones_v, sem).wait()
            return carry

        lax.fori_loop(0, NCH // DGRP, group, 0)
        plsc.subcore_barrier()
        pltpu.sync_copy(acc.at[pl.ds(s * ROWS_T, ROWS_T)],
                        out_hbm.at[c].at[pl.ds(s * ROWS_T, ROWS_T)])

    return k(dst_pad, ones_hbm, zeros16_hbm)


def _agg_partials(ht, src_pad, dst_pad, zeros_hbm):
    @functools.partial(
        pl.kernel,
        out_type=jax.ShapeDtypeStruct((NC, NPAD, D), jnp.float32),
        mesh=_sc_mesh(),
        scratch_types=[
            pltpu.VMEM((NCH, ECH), jnp.int32),
            pltpu.VMEM((NCH, ECH), jnp.int32),
            pltpu.VMEM((ECH, D), jnp.float32),
            pltpu.VMEM_SHARED((NPAD, D), jnp.float32),
            pltpu.SemaphoreType.DMA,
        ],
    )
    def k(ht_hbm, src_hbm, dst_hbm, zeros_ref, out_hbm,
          sidx2, didx2, rows, acc, sem):
        c = lax.axis_index("c")
        s = lax.axis_index("s")
        wid = s * NC + c
        pltpu.sync_copy(src_hbm.at[pl.ds(wid * NCH, NCH)], sidx2)
        pltpu.sync_copy(dst_hbm.at[pl.ds(wid * NCH, NCH)], didx2)
        pltpu.sync_copy(zeros_ref, acc.at[pl.ds(s * ROWS_T, ROWS_T)])
        plsc.subcore_barrier()

        def chunk(j, carry):
            pltpu.async_copy(ht_hbm.at[sidx2.at[j]], rows, sem).wait()
            pltpu.sync_copy(rows, acc.at[didx2.at[j]], add=True)
            return carry

        lax.fori_loop(0, NCH, chunk, 0)
        plsc.subcore_barrier()
        pltpu.sync_copy(acc.at[pl.ds(s * ROWS_T, ROWS_T)],
                        out_hbm.at[c].at[pl.ds(s * ROWS_T, ROWS_T)])

    return k(ht, src_pad, dst_pad, zeros_hbm)


def _enc_mm_body(x_ref, we_ref, be_ref, h_ref):
    h_ref[...] = jnp.dot(x_ref[...], we_ref[...],
                         preferred_element_type=jnp.float32) + be_ref[...]


def _enc_mm(x, W_enc, b_enc2):
    return pl.pallas_call(
        _enc_mm_body,
        grid=(GRID,),
        in_specs=[
            pl.BlockSpec((RB, D), lambda i: (i, 0)),
            pl.BlockSpec((D, D), lambda i: (0, 0)),
            pl.BlockSpec((1, D), lambda i: (0, 0)),
        ],
        out_specs=pl.BlockSpec((RB, D), lambda i: (i, 0)),
        out_shape=jax.ShapeDtypeStruct((N, D), jnp.float32),
    )(x, W_enc, b_enc2)


def _enc_scale_body(h_ref, dga_ref, dgb_ref, ht_ref, dv_ref):
    deg = dga_ref[:, 0:1] + dgb_ref[:, 0:1]
    dinv = 1.0 / jnp.sqrt(jnp.maximum(deg, 1.0))
    dinvb = jnp.broadcast_to(dinv, (RB, D))
    dv_ref[...] = dinvb
    ht_ref[...] = h_ref[...] * dinvb


def _enc_scale(h, degA, degB):
    return pl.pallas_call(
        _enc_scale_body,
        grid=(GRID,),
        in_specs=[
            pl.BlockSpec((RB, D), lambda i: (i, 0)),
            pl.BlockSpec((RB, DEGC), lambda i: (i, 0)),
            pl.BlockSpec((RB, DEGC), lambda i: (i, 0)),
        ],
        out_specs=[
            pl.BlockSpec((RB, D), lambda i: (i, 0)),
            pl.BlockSpec((RB, D), lambda i: (i, 0)),
        ],
        out_shape=[
            jax.ShapeDtypeStruct((N, D), jnp.float32),
            jax.ShapeDtypeStruct((N, D), jnp.float32),
        ],
    )(h, degA, degB)


def _conv_body(s0_ref, s1_ref, h_ref, dv_ref, w_ref, b_ref, hn_ref, htn_ref):
    dinvb = dv_ref[...]
    agg = (s0_ref[...] + s1_ref[...]) * dinvb
    z = jnp.dot(agg, w_ref[...], preferred_element_type=jnp.float32) + b_ref[...]
    h = jnp.maximum(z, 0.0) + h_ref[...]
    hn_ref[...] = h
    htn_ref[...] = h * dinvb


def _conv(S0, S1, h, dinvb, W_l, b_l2):
    return pl.pallas_call(
        _conv_body,
        grid=(GRID,),
        in_specs=[
            pl.BlockSpec((RB, D), lambda i: (i, 0)),
            pl.BlockSpec((RB, D), lambda i: (i, 0)),
            pl.BlockSpec((RB, D), lambda i: (i, 0)),
            pl.BlockSpec((RB, D), lambda i: (i, 0)),
            pl.BlockSpec((D, D), lambda i: (0, 0)),
            pl.BlockSpec((1, D), lambda i: (0, 0)),
        ],
        out_specs=[
            pl.BlockSpec((RB, D), lambda i: (i, 0)),
            pl.BlockSpec((RB, D), lambda i: (i, 0)),
        ],
        out_shape=[
            jax.ShapeDtypeStruct((N, D), jnp.float32),
            jax.ShapeDtypeStruct((N, D), jnp.float32),
        ],
    )(S0, S1, h, dinvb, W_l, b_l2)


def _vq_body(h_ref, cb_ref, ids_ref, com_ref):
    i = pl.program_id(0)

    @pl.when(i == 0)
    def _():
        com_ref[...] = jnp.zeros((1, 1), jnp.float32)

    resid = h_ref[...]
    kio = lax.broadcasted_iota(jnp.int32, (RB, K), 1)
    csum = jnp.float32(0.0)
    idxs = []
    for q in range(Q):
        cb = cb_ref[q]
        nrm = jnp.sqrt(jnp.sum(cb * cb, axis=1, keepdims=True)) + 1e-8
        cbn = cb / nrm
        sim = lax.dot_general(resid, cbn, (((1,), (1,)), ((), ())),
                              preferred_element_type=jnp.float32)
        mx = jnp.max(sim, axis=1, keepdims=True)
        idx = jnp.min(jnp.where(sim >= mx, kio, K), axis=1, keepdims=True)
        onehot = (kio == idx).astype(jnp.float32)
        quant = jnp.dot(onehot, cbn, preferred_element_type=jnp.float32)
        diff = quant - resid
        csum = csum + jnp.sum(diff * diff)
        resid = resid - quant
        idxs.append(idx.astype(jnp.float32))
    ids_ref[...] = jnp.concatenate(idxs, axis=1)
    com_ref[...] = com_ref[...] + csum * (0.25 / (N * D))


def _vq(h, cb_l):
    return pl.pallas_call(
        _vq_body,
        grid=(GRID,),
        in_specs=[
            pl.BlockSpec((RB, D), lambda i: (i, 0)),
            pl.BlockSpec((Q, K, D), lambda i: (0, 0, 0)),
        ],
        out_specs=[
            pl.BlockSpec((RB, Q), lambda i: (i, 0)),
            pl.BlockSpec((1, 1), lambda i: (0, 0)),
        ],
        out_shape=[
            jax.ShapeDtypeStruct((N, Q), jnp.float32),
            jax.ShapeDtypeStruct((1, 1), jnp.float32),
        ],
    )(h, cb_l)


def _pool_body(h_ref, ids_ref, bi_ref, wh_ref, bh_ref,
               out_ref, gid_ref, pacc, gacc, cacc):
    i = pl.program_id(0)

    @pl.when(i == 0)
    def _():
        pacc[...] = jnp.zeros_like(pacc)
        gacc[...] = jnp.zeros_like(gacc)
        cacc[...] = jnp.zeros_like(cacc)

    gio = lax.broadcasted_iota(jnp.int32, (RB, G), 1)
    oh = (bi_ref[...] == gio).astype(jnp.float32)
    pacc[...] += lax.dot_general(oh, h_ref[...], (((0,), (0,)), ((), ())),
                                 preferred_element_type=jnp.float32)
    gacc[...] += lax.dot_general(oh, ids_ref[...], (((0,), (0,)), ((), ())),
                                 preferred_element_type=jnp.float32)
    ones = jnp.ones((RB, 1), jnp.float32)
    cacc[...] += lax.dot_general(oh, ones, (((0,), (0,)), ((), ())),
                                 preferred_element_type=jnp.float32)

    @pl.when(i == pl.num_programs(0) - 1)
    def _():
        pooled = pacc[...] / jnp.maximum(cacc[...], 1.0)
        out_ref[...] = jnp.dot(pooled, wh_ref[...],
                               preferred_element_type=jnp.float32) + bh_ref[...]
        gid_ref[...] = gacc[...]


def _pool(h, ids_cat, bi2, W_head, b_head2):
    return pl.pallas_call(
        _pool_body,
        grid=(GRID,),
        in_specs=[
            pl.BlockSpec((RB, D), lambda i: (i, 0)),
            pl.BlockSpec((RB, L * Q), lambda i: (i, 0)),
            pl.BlockSpec((RB, 1), lambda i: (i, 0)),
            pl.BlockSpec((D, DOUT), lambda i: (0, 0)),
            pl.BlockSpec((1, DOUT), lambda i: (0, 0)),
        ],
        out_specs=[
            pl.BlockSpec((G, DOUT), lambda i: (0, 0)),
            pl.BlockSpec((G, L * Q), lambda i: (0, 0)),
        ],
        out_shape=[
            jax.ShapeDtypeStruct((G, DOUT), jnp.float32),
            jax.ShapeDtypeStruct((G, L * Q), jnp.float32),
        ],
        scratch_shapes=[
            pltpu.VMEM((G, D), jnp.float32),
            pltpu.VMEM((G, L * Q), jnp.float32),
            pltpu.VMEM((G, 1), jnp.float32),
        ],
    )(h, ids_cat, bi2, W_head, b_head2)


def kernel(x, edge_index, batch_idx, W_enc, b_enc, W_conv, b_conv,
           codebooks, W_head, b_head):
    src = edge_index[0]
    dst = edge_index[1]
    pad_e = EPAD - E
    src_pad = jnp.concatenate([src, jnp.zeros((pad_e,), jnp.int32)]).reshape(NW * NCH, ECH)
    dst_pad = jnp.concatenate([dst, jnp.full((pad_e,), N, jnp.int32)]).reshape(NW * NCH, ECH)
    ones_hbm = jnp.ones((ECH, DEGC), jnp.float32)
    zeros16_hbm = jnp.zeros((ROWS_T, DEGC), jnp.float32)
    zeros_hbm = jnp.zeros((ROWS_T, D), jnp.float32)

    degP = _deg_partials(dst_pad, ones_hbm, zeros16_hbm)
    h = _enc_mm(x, W_enc, b_enc[None, :])
    ht, dinvb = _enc_scale(h, degP[0, :N], degP[1, :N])

    commits = []
    ids_list = []
    for l in range(L):
        SP = _agg_partials(ht, src_pad, dst_pad, zeros_hbm)
        h, ht = _conv(SP[0, :N], SP[1, :N], h, dinvb,
                      W_conv[l], b_conv[l][None, :])
        ids_l, com_l = _vq(h, codebooks[l])
        commits.append(com_l)
        ids_list.append(ids_l)

    ids_cat = jnp.concatenate(ids_list, axis=1)
    out, graph_id = _pool(h, ids_cat, batch_idx[:, None],
                          W_head, b_head[None, :])
    total_commit = (commits[0] + commits[1] + commits[2] + commits[3])[0, 0]
    return out, total_commit, graph_id


# speedup vs baseline: 1.5614x; 1.1378x over previous
import functools

import jax
import jax.numpy as jnp
from jax import lax
from jax.experimental import pallas as pl
from jax.experimental.pallas import tpu as pltpu
from jax.experimental.pallas import tpu_sc as plsc

N = 10000
E = 320000
D = 128
L = 4
Q = 3
K = 16
G = 64
DOUT = 10

RB = 2000
GRID = N // RB

NC, NS = 2, 16
NW = NC * NS
ECH = 128
EPT = 10112
EPAD = EPT * NW
NCH = EPT // ECH
NPAD = 10240
ROWS_T = NPAD // NS
DEGC = 128


def _sc_mesh():
    return plsc.VectorSubcoreMesh(core_axis_name="c", subcore_axis_name="s")


def _deg_partials(dst_pad, ones_hbm, zeros16_hbm):
    @functools.partial(
        pl.kernel,
        out_type=jax.ShapeDtypeStruct((NC, NPAD, DEGC), jnp.float32),
        mesh=_sc_mesh(),
        scratch_types=[
            pltpu.VMEM((ECH,), jnp.int32),
            pltpu.VMEM((ECH, DEGC), jnp.float32),
            pltpu.VMEM_SHARED((NPAD, DEGC), jnp.float32),
        ],
    )
    def k(dst_hbm, ones_ref, zeros_ref, out_hbm, didx, ones_v, acc):
        c = lax.axis_index("c")
        s = lax.axis_index("s")
        wid = s * NC + c
        pltpu.sync_copy(ones_ref, ones_v)
        pltpu.sync_copy(zeros_ref, acc.at[pl.ds(s * ROWS_T, ROWS_T)])
        plsc.subcore_barrier()
        base = wid * EPT

        def chunk(j, carry):
            e0 = base + j * ECH
            pltpu.sync_copy(dst_hbm.at[pl.ds(e0, ECH)], didx)
            pltpu.sync_copy(ones_v, acc.at[didx], add=True)
            return carry

        lax.fori_loop(0, NCH, chunk, 0)
        plsc.subcore_barrier()
        pltpu.sync_copy(acc.at[pl.ds(s * ROWS_T, ROWS_T)],
                        out_hbm.at[c].at[pl.ds(s * ROWS_T, ROWS_T)])

    return k(dst_pad, ones_hbm, zeros16_hbm)


def _agg_partials(ht, src_pad, dst_pad, zeros_hbm):
    @functools.partial(
        pl.kernel,
        out_type=jax.ShapeDtypeStruct((NC, NPAD, D), jnp.float32),
        mesh=_sc_mesh(),
        scratch_types=[
            pltpu.VMEM((ECH,), jnp.int32),
            pltpu.VMEM((ECH,), jnp.int32),
            pltpu.VMEM((ECH, D), jnp.float32),
            pltpu.VMEM_SHARED((NPAD, D), jnp.float32),
            pltpu.SemaphoreType.DMA,
        ],
    )
    def k(ht_hbm, src_hbm, dst_hbm, zeros_ref, out_hbm,
          sidx, didx, rows, acc, sem):
        c = lax.axis_index("c")
        s = lax.axis_index("s")
        wid = s * NC + c
        pltpu.sync_copy(zeros_ref, acc.at[pl.ds(s * ROWS_T, ROWS_T)])
        plsc.subcore_barrier()
        base = wid * EPT

        def chunk(j, carry):
            e0 = base + j * ECH
            pltpu.sync_copy(src_hbm.at[pl.ds(e0, ECH)], sidx)
            pltpu.sync_copy(dst_hbm.at[pl.ds(e0, ECH)], didx)
            pltpu.async_copy(ht_hbm.at[sidx], rows, sem).wait()
            pltpu.sync_copy(rows, acc.at[didx], add=True)
            return carry

        lax.fori_loop(0, NCH, chunk, 0)
        plsc.subcore_barrier()
        pltpu.sync_copy(acc.at[pl.ds(s * ROWS_T, ROWS_T)],
                        out_hbm.at[c].at[pl.ds(s * ROWS_T, ROWS_T)])

    return k(ht, src_pad, dst_pad, zeros_hbm)


def _enc_body(x_ref, we_ref, be_ref, dga_ref, dgb_ref, h_ref, ht_ref, dv_ref):
    deg = dga_ref[:, 0:1] + dgb_ref[:, 0:1]
    dinv = 1.0 / jnp.sqrt(jnp.maximum(deg, 1.0))
    dinvb = jnp.broadcast_to(dinv, (RB, D))
    h = jnp.dot(x_ref[...], we_ref[...],
                preferred_element_type=jnp.float32) + be_ref[...]
    h_ref[...] = h
    dv_ref[...] = dinvb
    ht_ref[...] = h * dinvb


def _encode(x, W_enc, b_enc2, degA, degB):
    return pl.pallas_call(
        _enc_body,
        grid=(GRID,),
        in_specs=[
            pl.BlockSpec((RB, D), lambda i: (i, 0)),
            pl.BlockSpec((D, D), lambda i: (0, 0)),
            pl.BlockSpec((1, D), lambda i: (0, 0)),
            pl.BlockSpec((RB, DEGC), lambda i: (i, 0)),
            pl.BlockSpec((RB, DEGC), lambda i: (i, 0)),
        ],
        out_specs=[
            pl.BlockSpec((RB, D), lambda i: (i, 0)),
            pl.BlockSpec((RB, D), lambda i: (i, 0)),
            pl.BlockSpec((RB, D), lambda i: (i, 0)),
        ],
        out_shape=[
            jax.ShapeDtypeStruct((N, D), jnp.float32),
            jax.ShapeDtypeStruct((N, D), jnp.float32),
            jax.ShapeDtypeStruct((N, D), jnp.float32),
        ],
    )(x, W_enc, b_enc2, degA, degB)


def _layer_body(s0_ref, s1_ref, h_ref, dv_ref, w_ref, b_ref, cb_ref,
                hn_ref, htn_ref, ids_ref, com_ref):
    i = pl.program_id(0)
    dinvb = dv_ref[...]
    agg = (s0_ref[...] + s1_ref[...]) * dinvb
    z = jnp.dot(agg, w_ref[...], preferred_element_type=jnp.float32) + b_ref[...]
    h = jnp.maximum(z, 0.0) + h_ref[...]
    hn_ref[...] = h
    htn_ref[...] = h * dinvb

    @pl.when(i == 0)
    def _():
        com_ref[...] = jnp.zeros((1, 1), jnp.float32)

    resid = h
    kio = lax.broadcasted_iota(jnp.int32, (RB, K), 1)
    csum = jnp.float32(0.0)
    idxs = []
    for q in range(Q):
        cb = cb_ref[q]
        nrm = jnp.sqrt(jnp.sum(cb * cb, axis=1, keepdims=True)) + 1e-8
        cbn = cb / nrm
        sim = lax.dot_general(resid, cbn, (((1,), (1,)), ((), ())),
                              preferred_element_type=jnp.float32)
        mx = jnp.max(sim, axis=1, keepdims=True)
        idx = jnp.min(jnp.where(sim >= mx, kio, K), axis=1, keepdims=True)
        onehot = (kio == idx).astype(jnp.float32)
        quant = jnp.dot(onehot, cbn, preferred_element_type=jnp.float32)
        diff = quant - resid
        csum = csum + jnp.sum(diff * diff)
        resid = resid - quant
        idxs.append(idx.astype(jnp.float32))
    ids_ref[...] = jnp.concatenate(idxs, axis=1)
    com_ref[...] = com_ref[...] + csum * (0.25 / (N * D))


def _layer(S0, S1, h, dinvb, W_l, b_l2, cb_l):
    return pl.pallas_call(
        _layer_body,
        grid=(GRID,),
        in_specs=[
            pl.BlockSpec((RB, D), lambda i: (i, 0)),
            pl.BlockSpec((RB, D), lambda i: (i, 0)),
            pl.BlockSpec((RB, D), lambda i: (i, 0)),
            pl.BlockSpec((RB, D), lambda i: (i, 0)),
            pl.BlockSpec((D, D), lambda i: (0, 0)),
            pl.BlockSpec((1, D), lambda i: (0, 0)),
            pl.BlockSpec((Q, K, D), lambda i: (0, 0, 0)),
        ],
        out_specs=[
            pl.BlockSpec((RB, D), lambda i: (i, 0)),
            pl.BlockSpec((RB, D), lambda i: (i, 0)),
            pl.BlockSpec((RB, Q), lambda i: (i, 0)),
            pl.BlockSpec((1, 1), lambda i: (0, 0)),
        ],
        out_shape=[
            jax.ShapeDtypeStruct((N, D), jnp.float32),
            jax.ShapeDtypeStruct((N, D), jnp.float32),
            jax.ShapeDtypeStruct((N, Q), jnp.float32),
            jax.ShapeDtypeStruct((1, 1), jnp.float32),
        ],
    )(S0, S1, h, dinvb, W_l, b_l2, cb_l)


def _pool_body(h_ref, ids_ref, bi_ref, wh_ref, bh_ref,
               out_ref, gid_ref, pacc, gacc, cacc):
    i = pl.program_id(0)

    @pl.when(i == 0)
    def _():
        pacc[...] = jnp.zeros_like(pacc)
        gacc[...] = jnp.zeros_like(gacc)
        cacc[...] = jnp.zeros_like(cacc)

    gio = lax.broadcasted_iota(jnp.int32, (RB, G), 1)
    oh = (bi_ref[...] == gio).astype(jnp.float32)
    pacc[...] += lax.dot_general(oh, h_ref[...], (((0,), (0,)), ((), ())),
                                 preferred_element_type=jnp.float32)
    gacc[...] += lax.dot_general(oh, ids_ref[...], (((0,), (0,)), ((), ())),
                                 preferred_element_type=jnp.float32)
    ones = jnp.ones((RB, 1), jnp.float32)
    cacc[...] += lax.dot_general(oh, ones, (((0,), (0,)), ((), ())),
                                 preferred_element_type=jnp.float32)

    @pl.when(i == pl.num_programs(0) - 1)
    def _():
        pooled = pacc[...] / jnp.maximum(cacc[...], 1.0)
        out_ref[...] = jnp.dot(pooled, wh_ref[...],
                               preferred_element_type=jnp.float32) + bh_ref[...]
        gid_ref[...] = gacc[...]


def _pool(h, ids_cat, bi2, W_head, b_head2):
    return pl.pallas_call(
        _pool_body,
        grid=(GRID,),
        in_specs=[
            pl.BlockSpec((RB, D), lambda i: (i, 0)),
            pl.BlockSpec((RB, L * Q), lambda i: (i, 0)),
            pl.BlockSpec((RB, 1), lambda i: (i, 0)),
            pl.BlockSpec((D, DOUT), lambda i: (0, 0)),
            pl.BlockSpec((1, DOUT), lambda i: (0, 0)),
        ],
        out_specs=[
            pl.BlockSpec((G, DOUT), lambda i: (0, 0)),
            pl.BlockSpec((G, L * Q), lambda i: (0, 0)),
        ],
        out_shape=[
            jax.ShapeDtypeStruct((G, DOUT), jnp.float32),
            jax.ShapeDtypeStruct((G, L * Q), jnp.float32),
        ],
        scratch_shapes=[
            pltpu.VMEM((G, D), jnp.float32),
            pltpu.VMEM((G, L * Q), jnp.float32),
            pltpu.VMEM((G, 1), jnp.float32),
        ],
    )(h, ids_cat, bi2, W_head, b_head2)


def kernel(x, edge_index, batch_idx, W_enc, b_enc, W_conv, b_conv,
           codebooks, W_head, b_head):
    src = edge_index[0]
    dst = edge_index[1]
    pad_e = EPAD - E
    src_pad = jnp.concatenate([src, jnp.zeros((pad_e,), jnp.int32)])
    dst_pad = jnp.concatenate([dst, jnp.full((pad_e,), N, jnp.int32)])
    ones_hbm = jnp.ones((ECH, DEGC), jnp.float32)
    zeros16_hbm = jnp.zeros((ROWS_T, DEGC), jnp.float32)
    zeros_hbm = jnp.zeros((ROWS_T, D), jnp.float32)

    degP = _deg_partials(dst_pad, ones_hbm, zeros16_hbm)
    h, ht, dinvb = _encode(x, W_enc, b_enc[None, :], degP[0, :N], degP[1, :N])

    commits = []
    ids_list = []
    for l in range(L):
        SP = _agg_partials(ht, src_pad, dst_pad, zeros_hbm)
        h, ht, ids_l, com_l = _layer(SP[0, :N], SP[1, :N], h, dinvb,
                                     W_conv[l], b_conv[l][None, :],
                                     codebooks[l])
        commits.append(com_l)
        ids_list.append(ids_l)

    ids_cat = jnp.concatenate(ids_list, axis=1)
    out, graph_id = _pool(h, ids_cat, batch_idx[:, None],
                          W_head, b_head[None, :])
    total_commit = (commits[0] + commits[1] + commits[2] + commits[3])[0, 0]
    return out, total_commit, graph_id
